# Initial kernel scaffold; baseline (speedup 1.0000x reference)
import functools

import jax
import jax.numpy as jnp
from jax import lax
from jax.experimental import pallas as pl
from jax.experimental.pallas import tpu as pltpu
from jax.experimental.pallas import tpu_sc as plsc

N_NODE = 10000
N_PAD = 10240
NC, NS = 2, 16
CHUNK = 128
ROWS_1D = N_PAD // NS
N_SPREAD = 64

E_INT = 320000
E_SIM = 160000
NCH_I = -(-E_INT // (NS * CHUNK))
NCH_S = -(-E_SIM // (NS * CHUNK))

_mesh = plsc.VectorSubcoreMesh(core_axis_name="c", subcore_axis_name="s")


def _pad3(row, nch):
    tot = NS * nch * CHUNK
    pad = tot - row.shape[0]
    fill = N_NODE + (jnp.arange(pad, dtype=jnp.int32) % N_SPREAD)
    return jnp.concatenate([row.astype(jnp.int32), fill]).reshape(NS, nch, CHUNK)


@functools.partial(
    pl.kernel,
    out_type=[jax.ShapeDtypeStruct((N_PAD,), jnp.float32)] * 6,
    mesh=_mesh,
    scratch_types=[
        pltpu.VMEM((NCH_I, CHUNK), jnp.int32),
        pltpu.VMEM((CHUNK,), jnp.float32),
        pltpu.VMEM_SHARED((N_PAD,), jnp.float32),
        pltpu.VMEM_SHARED((N_PAD,), jnp.float32),
        pltpu.VMEM_SHARED((N_PAD,), jnp.float32),
    ],
)
def _degree_kernel(gi_s3, gi_d3, ds_s3, ds_d3, as_s3, as_d3, zeros1,
                   o_gis, o_gid, o_as, o_dss, o_dsd, o_ad,
                   idx_v, ones_v, acc_a, acc_b, acc_c):
    t = lax.axis_index("s")
    c = lax.axis_index("c")
    sl = pl.ds(t * ROWS_1D, ROWS_1D)
    pltpu.sync_copy(zeros1.at[sl], acc_a.at[sl])
    pltpu.sync_copy(zeros1.at[sl], acc_b.at[sl])
    pltpu.sync_copy(zeros1.at[sl], acc_c.at[sl])
    for i in range(CHUNK // 16):
        ones_v[pl.ds(i * 16, 16)] = jnp.ones((16,), jnp.float32)
    plsc.subcore_barrier()

    def run(idx3, nch, acc):
        pltpu.sync_copy(idx3.at[t], idx_v.at[pl.ds(0, nch)])

        def body(j, carry):
            pltpu.sync_copy(ones_v, acc.at[idx_v.at[j]], add=True)
            return carry

        lax.fori_loop(0, nch, body, 0)

    @pl.when(c == 0)
    def _():
        run(gi_s3, NCH_I, acc_a)
        run(gi_d3, NCH_I, acc_b)
        run(as_s3, NCH_S, acc_c)

    @pl.when(c == 1)
    def _():
        run(ds_s3, NCH_S, acc_a)
        run(ds_d3, NCH_S, acc_b)
        run(as_d3, NCH_S, acc_c)

    plsc.subcore_barrier()

    @pl.when(c == 0)
    def _():
        pltpu.sync_copy(acc_a.at[sl], o_gis.at[sl])
        pltpu.sync_copy(acc_b.at[sl], o_gid.at[sl])
        pltpu.sync_copy(acc_c.at[sl], o_as.at[sl])

    @pl.when(c == 1)
    def _():
        pltpu.sync_copy(acc_a.at[sl], o_dss.at[sl])
        pltpu.sync_copy(acc_b.at[sl], o_dsd.at[sl])
        pltpu.sync_copy(acc_c.at[sl], o_ad.at[sl])


def _make_scatter_kernel(width):
    @functools.partial(
        pl.kernel,
        out_type=[jax.ShapeDtypeStruct((N_PAD, width), jnp.float32)] * 4,
        mesh=_mesh,
        scratch_types=[
            pltpu.VMEM((NCH_I, CHUNK), jnp.int32),
            pltpu.VMEM((NCH_I, CHUNK), jnp.int32),
            pltpu.VMEM((CHUNK, width), jnp.float32),
            pltpu.SemaphoreType.DMA,
            pltpu.VMEM_SHARED((N_PAD, width), jnp.float32),
            pltpu.VMEM_SHARED((N_PAD, width), jnp.float32),
        ],
    )
    def scatter_kernel(hs_ii, hs_dg, hs_dd, hs_gd,
                       gi_s3, gi_d3, ds_s3, ds_d3, as_s3, as_d3, zeros2,
                       o_ii, o_dg, o_dd, o_gd,
                       src_v, dst_v, rows_v, sem, acc0, acc1):
        t = lax.axis_index("s")
        c = lax.axis_index("c")
        rsl = pl.ds(t * ROWS_1D, ROWS_1D)
        pltpu.sync_copy(zeros2.at[rsl], acc0.at[rsl])
        pltpu.sync_copy(zeros2.at[rsl], acc1.at[rsl])
        plsc.subcore_barrier()

        def run(tbl, s3, d3, nch, acc):
            pltpu.sync_copy(s3.at[t], src_v.at[pl.ds(0, nch)])
            pltpu.sync_copy(d3.at[t], dst_v.at[pl.ds(0, nch)])

            def body(j, carry):
                pltpu.async_copy(tbl.at[src_v.at[j]], rows_v, sem).wait()
                pltpu.sync_copy(rows_v, acc.at[dst_v.at[j]], add=True)
                return carry

            lax.fori_loop(0, nch, body, 0)

        @pl.when(c == 0)
        def _():
            run(hs_ii, gi_s3, gi_d3, NCH_I, acc0)
            run(hs_dg, as_d3, as_s3, NCH_S, acc1)

        @pl.when(c == 1)
        def _():
            run(hs_dd, ds_s3, ds_d3, NCH_S, acc0)
            run(hs_gd, as_s3, as_d3, NCH_S, acc1)

        plsc.subcore_barrier()

        @pl.when(c == 0)
        def _():
            pltpu.sync_copy(acc0.at[rsl], o_ii.at[rsl])
            pltpu.sync_copy(acc1.at[rsl], o_dg.at[rsl])

        @pl.when(c == 1)
        def _():
            pltpu.sync_copy(acc0.at[rsl], o_dd.at[rsl])
            pltpu.sync_copy(acc1.at[rsl], o_gd.at[rsl])

    return scatter_kernel


_scatter64 = _make_scatter_kernel(64)
_scatter32 = _make_scatter_kernel(32)


BLK = 1024
_GRID = N_PAD // BLK


def _rowmm(x, w):
    return lax.dot_general(x, w, (((1,), (1,)), ((), ())),
                           preferred_element_type=jnp.float32)


def _prep1_body(xg, xd, w_ii, w_gd, w_dd, w_dg,
                c_gis, c_as, c_dss, c_ad,
                hs_ii, hs_gd, hs_dd, hs_dg):
    a_gis = lax.rsqrt(c_gis[...] + 1.0)[:, None]
    a_as = lax.rsqrt(jnp.maximum(c_as[...], 1.0))[:, None]
    a_dss = lax.rsqrt(c_dss[...] + 1.0)[:, None]
    a_ad = lax.rsqrt(jnp.maximum(c_ad[...], 1.0))[:, None]
    hs_ii[...] = _rowmm(xg[...], w_ii[...]) * a_gis
    hs_gd[...] = _rowmm(xg[...], w_gd[...]) * a_as
    hs_dd[...] = _rowmm(xd[...], w_dd[...]) * a_dss
    hs_dg[...] = _rowmm(xd[...], w_dg[...]) * a_ad


def _combine_body(m_ii, m_dg, m_dd, m_gd, hs_ii, hs_dd,
                  c_gis, c_gid, c_as, c_dss, c_dsd, c_ad,
                  bias_g, bias_d, w_ii, w_gd, w_dd, w_dg,
                  o_ii, o_gd, o_dd, o_dg):
    a_gis = lax.rsqrt(c_gis[...] + 1.0)[:, None]
    a_gid = lax.rsqrt(c_gid[...] + 1.0)[:, None]
    a_as = lax.rsqrt(jnp.maximum(c_as[...], 1.0))[:, None]
    a_dss = lax.rsqrt(c_dss[...] + 1.0)[:, None]
    a_dsd = lax.rsqrt(c_dsd[...] + 1.0)[:, None]
    a_ad = lax.rsqrt(jnp.maximum(c_ad[...], 1.0))[:, None]
    g = jnp.maximum(a_gid * (m_ii[...] + hs_ii[...]) + a_as * m_dg[...]
                    + bias_g[...], 0.0)
    d = jnp.maximum(a_dsd * (m_dd[...] + hs_dd[...]) + a_ad * m_gd[...]
                    + bias_d[...], 0.0)
    o_ii[...] = _rowmm(g, w_ii[...]) * a_gis
    o_gd[...] = _rowmm(g, w_gd[...]) * a_as
    o_dd[...] = _rowmm(d, w_dd[...]) * a_dss
    o_dg[...] = _rowmm(d, w_dg[...]) * a_ad


def _final_body(m_ii, m_dg, m_dd, m_gd, hs_ii, hs_dd,
                c_gid, c_as, c_dsd, c_ad, bias_g, bias_d,
                o_g, o_d):
    a_gid = lax.rsqrt(c_gid[...] + 1.0)[:, None]
    a_as = lax.rsqrt(jnp.maximum(c_as[...], 1.0))[:, None]
    a_dsd = lax.rsqrt(c_dsd[...] + 1.0)[:, None]
    a_ad = lax.rsqrt(jnp.maximum(c_ad[...], 1.0))[:, None]
    o_g[...] = jnp.maximum(a_gid * (m_ii[...] + hs_ii[...]) + a_as * m_dg[...]
                           + bias_g[...], 0.0)
    o_d[...] = jnp.maximum(a_dsd * (m_dd[...] + hs_dd[...]) + a_ad * m_gd[...]
                           + bias_d[...], 0.0)


def _row_spec(width):
    if width is None:
        return pl.BlockSpec((BLK,), lambda i: (i,))
    return pl.BlockSpec((BLK, width), lambda i: (i, 0))


def _full_spec(shape):
    return pl.BlockSpec(shape, lambda i: tuple(0 for _ in shape))


def _tc_call(body, in_arrays, in_specs, out_widths):
    return pl.pallas_call(
        body,
        grid=(_GRID,),
        in_specs=in_specs,
        out_specs=[_row_spec(w) for w in out_widths],
        out_shape=[jax.ShapeDtypeStruct((N_PAD, w), jnp.float32)
                   for w in out_widths],
    )(*in_arrays)


def kernel(x_gene, x_disease, edge_interacts, edge_similar, edge_assoc,
           W1_ii, b1_ii, W1_dd, b1_dd, W1_gd, b1_gd, W1_dg, b1_dg,
           W2_ii, b2_ii, W2_dd, b2_dd, W2_gd, b2_gd, W2_dg, b2_dg):
    pad_rows = N_PAD - N_NODE
    xg = jnp.pad(x_gene.astype(jnp.float32), ((0, pad_rows), (0, 0)))
    xd = jnp.pad(x_disease.astype(jnp.float32), ((0, pad_rows), (0, 0)))

    gi_s3 = _pad3(edge_interacts[0], NCH_I)
    gi_d3 = _pad3(edge_interacts[1], NCH_I)
    ds_s3 = _pad3(edge_similar[0], NCH_S)
    ds_d3 = _pad3(edge_similar[1], NCH_S)
    as_s3 = _pad3(edge_assoc[0], NCH_S)
    as_d3 = _pad3(edge_assoc[1], NCH_S)

    zeros1 = jnp.zeros((N_PAD,), jnp.float32)
    zeros64 = jnp.zeros((N_PAD, 64), jnp.float32)
    zeros32 = jnp.zeros((N_PAD, 32), jnp.float32)

    c_gis, c_gid, c_as, c_dss, c_dsd, c_ad = _degree_kernel(
        gi_s3, gi_d3, ds_s3, ds_d3, as_s3, as_d3, zeros1)

    row = _row_spec
    full = _full_spec

    hs1_ii, hs1_gd, hs1_dd, hs1_dg = _tc_call(
        _prep1_body,
        [xg, xd, W1_ii, W1_gd, W1_dd, W1_dg, c_gis, c_as, c_dss, c_ad],
        [row(128), row(128),
         full((64, 128)), full((64, 128)), full((64, 128)), full((64, 128)),
         row(None), row(None), row(None), row(None)],
        [64, 64, 64, 64])

    m1_ii, m1_dg, m1_dd, m1_gd = _scatter64(
        hs1_ii, hs1_dg, hs1_dd, hs1_gd,
        gi_s3, gi_d3, ds_s3, ds_d3, as_s3, as_d3, zeros64)

    hs2_ii, hs2_gd, hs2_dd, hs2_dg = _tc_call(
        _combine_body,
        [m1_ii, m1_dg, m1_dd, m1_gd, hs1_ii, hs1_dd,
         c_gis, c_gid, c_as, c_dss, c_dsd, c_ad,
         (b1_ii + b1_dg).reshape(1, 64), (b1_dd + b1_gd).reshape(1, 64),
         W2_ii, W2_gd, W2_dd, W2_dg],
        [row(64)] * 6 + [row(None)] * 6
        + [full((1, 64)), full((1, 64)),
           full((32, 64)), full((32, 64)), full((32, 64)), full((32, 64))],
        [32, 32, 32, 32])

    m2_ii, m2_dg, m2_dd, m2_gd = _scatter32(
        hs2_ii, hs2_dg, hs2_dd, hs2_gd,
        gi_s3, gi_d3, ds_s3, ds_d3, as_s3, as_d3, zeros32)

    g2, d2 = _tc_call(
        _final_body,
        [m2_ii, m2_dg, m2_dd, m2_gd, hs2_ii, hs2_dd,
         c_gid, c_as, c_dsd, c_ad,
         (b2_ii + b2_dg).reshape(1, 32), (b2_dd + b2_gd).reshape(1, 32)],
        [row(32)] * 6 + [row(None)] * 4 + [full((1, 32)), full((1, 32))],
        [32, 32])

    return (g2[:N_NODE], d2[:N_NODE])


# speedup vs baseline: 19.3989x; 19.3989x over previous
import functools

import jax
import jax.numpy as jnp
from jax import lax
from jax.experimental import pallas as pl
from jax.experimental.pallas import tpu as pltpu
from jax.experimental.pallas import tpu_sc as plsc

N_NODE = 10000
N_PAD = 10240
NC, NS = 2, 16
CHUNK = 128
ROWS_1D = N_PAD // NS
N_SPREAD = 64

E_INT = 320000
E_SIM = 160000
NCH_I = -(-E_INT // (NS * CHUNK))
NCH_S = -(-E_SIM // (NS * CHUNK))

_mesh = plsc.VectorSubcoreMesh(core_axis_name="c", subcore_axis_name="s")
_sc_params = pltpu.CompilerParams(use_tc_tiling_on_sc=False)


def _pad3(row, nch):
    tot = NS * nch * CHUNK
    pad = tot - row.shape[0]
    fill = N_NODE + (jnp.arange(pad, dtype=jnp.int32) % N_SPREAD)
    return jnp.concatenate([row.astype(jnp.int32), fill]).reshape(NS, nch, CHUNK)


@functools.partial(
    pl.kernel,
    out_type=[jax.ShapeDtypeStruct((N_PAD,), jnp.float32)] * 6,
    mesh=_mesh,
    compiler_params=_sc_params,
    scratch_types=[
        pltpu.VMEM((NCH_I, CHUNK), jnp.int32),
        pltpu.VMEM((CHUNK,), jnp.float32),
        pltpu.VMEM_SHARED((N_PAD,), jnp.float32),
        pltpu.VMEM_SHARED((N_PAD,), jnp.float32),
        pltpu.VMEM_SHARED((N_PAD,), jnp.float32),
    ],
)
def _degree_kernel(gi_s3, gi_d3, ds_s3, ds_d3, as_s3, as_d3, zeros1,
                   o_gis, o_gid, o_as, o_dss, o_dsd, o_ad,
                   idx_v, ones_v, acc_a, acc_b, acc_c):
    t = lax.axis_index("s")
    c = lax.axis_index("c")
    sl = pl.ds(t * ROWS_1D, ROWS_1D)
    pltpu.sync_copy(zeros1.at[sl], acc_a.at[sl])
    pltpu.sync_copy(zeros1.at[sl], acc_b.at[sl])
    pltpu.sync_copy(zeros1.at[sl], acc_c.at[sl])
    for i in range(CHUNK // 16):
        ones_v[pl.ds(i * 16, 16)] = jnp.ones((16,), jnp.float32)
    plsc.subcore_barrier()

    def run(idx3, nch, acc):
        pltpu.sync_copy(idx3.at[t], idx_v.at[pl.ds(0, nch)])

        def body(j, carry):
            pltpu.sync_copy(ones_v, acc.at[idx_v.at[j]], add=True)
            return carry

        lax.fori_loop(0, nch, body, 0)

    @pl.when(c == 0)
    def _():
        run(gi_s3, NCH_I, acc_a)
        run(gi_d3, NCH_I, acc_b)
        run(as_s3, NCH_S, acc_c)

    @pl.when(c == 1)
    def _():
        run(ds_s3, NCH_S, acc_a)
        run(ds_d3, NCH_S, acc_b)
        run(as_d3, NCH_S, acc_c)

    plsc.subcore_barrier()

    @pl.when(c == 0)
    def _():
        pltpu.sync_copy(acc_a.at[sl], o_gis.at[sl])
        pltpu.sync_copy(acc_b.at[sl], o_gid.at[sl])
        pltpu.sync_copy(acc_c.at[sl], o_as.at[sl])

    @pl.when(c == 1)
    def _():
        pltpu.sync_copy(acc_a.at[sl], o_dss.at[sl])
        pltpu.sync_copy(acc_b.at[sl], o_dsd.at[sl])
        pltpu.sync_copy(acc_c.at[sl], o_ad.at[sl])


def _make_scatter_kernel(width):
    @functools.partial(
        pl.kernel,
        out_type=[jax.ShapeDtypeStruct((N_PAD, width), jnp.float32)] * 4,
        mesh=_mesh,
        compiler_params=_sc_params,
        scratch_types=[
            pltpu.VMEM((NCH_I, CHUNK), jnp.int32),
            pltpu.VMEM((NCH_I, CHUNK), jnp.int32),
            pltpu.VMEM((CHUNK, width), jnp.float32),
            pltpu.SemaphoreType.DMA,
            pltpu.VMEM_SHARED((N_PAD, width), jnp.float32),
            pltpu.VMEM_SHARED((N_PAD, width), jnp.float32),
        ],
    )
    def scatter_kernel(hs_ii, hs_dg, hs_dd, hs_gd,
                       gi_s3, gi_d3, ds_s3, ds_d3, as_s3, as_d3, zeros2,
                       o_ii, o_dg, o_dd, o_gd,
                       src_v, dst_v, rows_v, sem, acc0, acc1):
        t = lax.axis_index("s")
        c = lax.axis_index("c")
        rsl = pl.ds(t * ROWS_1D, ROWS_1D)
        pltpu.sync_copy(zeros2.at[rsl], acc0.at[rsl])
        pltpu.sync_copy(zeros2.at[rsl], acc1.at[rsl])
        plsc.subcore_barrier()

        def run(tbl, s3, d3, nch, acc):
            pltpu.sync_copy(s3.at[t], src_v.at[pl.ds(0, nch)])
            pltpu.sync_copy(d3.at[t], dst_v.at[pl.ds(0, nch)])

            def body(j, carry):
                pltpu.async_copy(tbl.at[src_v.at[j]], rows_v, sem).wait()
                pltpu.sync_copy(rows_v, acc.at[dst_v.at[j]], add=True)
                return carry

            lax.fori_loop(0, nch, body, 0)

        @pl.when(c == 0)
        def _():
            run(hs_ii, gi_s3, gi_d3, NCH_I, acc0)
            run(hs_dg, as_d3, as_s3, NCH_S, acc1)

        @pl.when(c == 1)
        def _():
            run(hs_dd, ds_s3, ds_d3, NCH_S, acc0)
            run(hs_gd, as_s3, as_d3, NCH_S, acc1)

        plsc.subcore_barrier()

        @pl.when(c == 0)
        def _():
            pltpu.sync_copy(acc0.at[rsl], o_ii.at[rsl])
            pltpu.sync_copy(acc1.at[rsl], o_dg.at[rsl])

        @pl.when(c == 1)
        def _():
            pltpu.sync_copy(acc0.at[rsl], o_dd.at[rsl])
            pltpu.sync_copy(acc1.at[rsl], o_gd.at[rsl])

    return scatter_kernel


_scatter64 = _make_scatter_kernel(64)
_scatter32 = _make_scatter_kernel(32)


BLK = 1024
_GRID = N_PAD // BLK


def _rowmm(x, w):
    return lax.dot_general(x, w, (((1,), (1,)), ((), ())),
                           preferred_element_type=jnp.float32)


def _prep1_body(xg, xd, w_ii, w_gd, w_dd, w_dg,
                c_gis, c_as, c_dss, c_ad,
                hs_ii, hs_gd, hs_dd, hs_dg):
    a_gis = lax.rsqrt(c_gis[...] + 1.0)[:, None]
    a_as = lax.rsqrt(jnp.maximum(c_as[...], 1.0))[:, None]
    a_dss = lax.rsqrt(c_dss[...] + 1.0)[:, None]
    a_ad = lax.rsqrt(jnp.maximum(c_ad[...], 1.0))[:, None]
    hs_ii[...] = _rowmm(xg[...], w_ii[...]) * a_gis
    hs_gd[...] = _rowmm(xg[...], w_gd[...]) * a_as
    hs_dd[...] = _rowmm(xd[...], w_dd[...]) * a_dss
    hs_dg[...] = _rowmm(xd[...], w_dg[...]) * a_ad


def _combine_body(m_ii, m_dg, m_dd, m_gd, hs_ii, hs_dd,
                  c_gis, c_gid, c_as, c_dss, c_dsd, c_ad,
                  bias_g, bias_d, w_ii, w_gd, w_dd, w_dg,
                  o_ii, o_gd, o_dd, o_dg):
    a_gis = lax.rsqrt(c_gis[...] + 1.0)[:, None]
    a_gid = lax.rsqrt(c_gid[...] + 1.0)[:, None]
    a_as = lax.rsqrt(jnp.maximum(c_as[...], 1.0))[:, None]
    a_dss = lax.rsqrt(c_dss[...] + 1.0)[:, None]
    a_dsd = lax.rsqrt(c_dsd[...] + 1.0)[:, None]
    a_ad = lax.rsqrt(jnp.maximum(c_ad[...], 1.0))[:, None]
    g = jnp.maximum(a_gid * (m_ii[...] + hs_ii[...]) + a_as * m_dg[...]
                    + bias_g[...], 0.0)
    d = jnp.maximum(a_dsd * (m_dd[...] + hs_dd[...]) + a_ad * m_gd[...]
                    + bias_d[...], 0.0)
    o_ii[...] = _rowmm(g, w_ii[...]) * a_gis
    o_gd[...] = _rowmm(g, w_gd[...]) * a_as
    o_dd[...] = _rowmm(d, w_dd[...]) * a_dss
    o_dg[...] = _rowmm(d, w_dg[...]) * a_ad


def _final_body(m_ii, m_dg, m_dd, m_gd, hs_ii, hs_dd,
                c_gid, c_as, c_dsd, c_ad, bias_g, bias_d,
                o_g, o_d):
    a_gid = lax.rsqrt(c_gid[...] + 1.0)[:, None]
    a_as = lax.rsqrt(jnp.maximum(c_as[...], 1.0))[:, None]
    a_dsd = lax.rsqrt(c_dsd[...] + 1.0)[:, None]
    a_ad = lax.rsqrt(jnp.maximum(c_ad[...], 1.0))[:, None]
    o_g[...] = jnp.maximum(a_gid * (m_ii[...] + hs_ii[...]) + a_as * m_dg[...]
                           + bias_g[...], 0.0)
    o_d[...] = jnp.maximum(a_dsd * (m_dd[...] + hs_dd[...]) + a_ad * m_gd[...]
                           + bias_d[...], 0.0)


def _row_spec(width):
    if width is None:
        return pl.BlockSpec((BLK,), lambda i: (i,))
    return pl.BlockSpec((BLK, width), lambda i: (i, 0))


def _full_spec(shape):
    return pl.BlockSpec(shape, lambda i: tuple(0 for _ in shape))


def _tc_call(body, in_arrays, in_specs, out_widths):
    return pl.pallas_call(
        body,
        grid=(_GRID,),
        in_specs=in_specs,
        out_specs=[_row_spec(w) for w in out_widths],
        out_shape=[jax.ShapeDtypeStruct((N_PAD, w), jnp.float32)
                   for w in out_widths],
    )(*in_arrays)


def kernel(x_gene, x_disease, edge_interacts, edge_similar, edge_assoc,
           W1_ii, b1_ii, W1_dd, b1_dd, W1_gd, b1_gd, W1_dg, b1_dg,
           W2_ii, b2_ii, W2_dd, b2_dd, W2_gd, b2_gd, W2_dg, b2_dg):
    pad_rows = N_PAD - N_NODE
    xg = jnp.pad(x_gene.astype(jnp.float32), ((0, pad_rows), (0, 0)))
    xd = jnp.pad(x_disease.astype(jnp.float32), ((0, pad_rows), (0, 0)))

    gi_s3 = _pad3(edge_interacts[0], NCH_I)
    gi_d3 = _pad3(edge_interacts[1], NCH_I)
    ds_s3 = _pad3(edge_similar[0], NCH_S)
    ds_d3 = _pad3(edge_similar[1], NCH_S)
    as_s3 = _pad3(edge_assoc[0], NCH_S)
    as_d3 = _pad3(edge_assoc[1], NCH_S)

    zeros1 = jnp.zeros((N_PAD,), jnp.float32)
    zeros64 = jnp.zeros((N_PAD, 64), jnp.float32)
    zeros32 = jnp.zeros((N_PAD, 32), jnp.float32)

    c_gis, c_gid, c_as, c_dss, c_dsd, c_ad = _degree_kernel(
        gi_s3, gi_d3, ds_s3, ds_d3, as_s3, as_d3, zeros1)

    row = _row_spec
    full = _full_spec

    hs1_ii, hs1_gd, hs1_dd, hs1_dg = _tc_call(
        _prep1_body,
        [xg, xd, W1_ii, W1_gd, W1_dd, W1_dg, c_gis, c_as, c_dss, c_ad],
        [row(128), row(128),
         full((64, 128)), full((64, 128)), full((64, 128)), full((64, 128)),
         row(None), row(None), row(None), row(None)],
        [64, 64, 64, 64])

    m1_ii, m1_dg, m1_dd, m1_gd = _scatter64(
        hs1_ii, hs1_dg, hs1_dd, hs1_gd,
        gi_s3, gi_d3, ds_s3, ds_d3, as_s3, as_d3, zeros64)

    hs2_ii, hs2_gd, hs2_dd, hs2_dg = _tc_call(
        _combine_body,
        [m1_ii, m1_dg, m1_dd, m1_gd, hs1_ii, hs1_dd,
         c_gis, c_gid, c_as, c_dss, c_dsd, c_ad,
         (b1_ii + b1_dg).reshape(1, 64), (b1_dd + b1_gd).reshape(1, 64),
         W2_ii, W2_gd, W2_dd, W2_dg],
        [row(64)] * 6 + [row(None)] * 6
        + [full((1, 64)), full((1, 64)),
           full((32, 64)), full((32, 64)), full((32, 64)), full((32, 64))],
        [32, 32, 32, 32])

    m2_ii, m2_dg, m2_dd, m2_gd = _scatter32(
        hs2_ii, hs2_dg, hs2_dd, hs2_gd,
        gi_s3, gi_d3, ds_s3, ds_d3, as_s3, as_d3, zeros32)

    g2, d2 = _tc_call(
        _final_body,
        [m2_ii, m2_dg, m2_dd, m2_gd, hs2_ii, hs2_dd,
         c_gid, c_as, c_dsd, c_ad,
         (b2_ii + b2_dg).reshape(1, 32), (b2_dd + b2_gd).reshape(1, 32)],
        [row(32)] * 6 + [row(None)] * 4 + [full((1, 32)), full((1, 32))],
        [32, 32])

    return (g2[:N_NODE], d2[:N_NODE])


# speedup vs baseline: 24.5113x; 1.2635x over previous
import functools

import jax
import jax.numpy as jnp
from jax import lax
from jax.experimental import pallas as pl
from jax.experimental.pallas import tpu as pltpu
from jax.experimental.pallas import tpu_sc as plsc

N_NODE = 10000
N_PAD = 10240
NC, NS = 2, 16
CHUNK = 128
ROWS_1D = N_PAD // NS
N_SPREAD = 64

E_INT = 320000
E_SIM = 160000
BLK_CH = 40
NCH_I = 160
NCH_S = 80

_mesh = plsc.VectorSubcoreMesh(core_axis_name="c", subcore_axis_name="s")
_sc_params = pltpu.CompilerParams(use_tc_tiling_on_sc=False)


def _pad3(row, nch):
    tot = NS * nch * CHUNK
    pad = tot - row.shape[0]
    fill = N_NODE + (jnp.arange(pad, dtype=jnp.int32) % N_SPREAD)
    return jnp.concatenate([row.astype(jnp.int32), fill]).reshape(NS, nch, CHUNK)


@functools.partial(
    pl.kernel,
    out_type=[jax.ShapeDtypeStruct((N_PAD,), jnp.float32)] * 6,
    mesh=_mesh,
    compiler_params=_sc_params,
    scratch_types=[
        pltpu.VMEM((NCH_I, CHUNK), jnp.int32),
        pltpu.VMEM((CHUNK,), jnp.float32),
        pltpu.VMEM_SHARED((N_PAD,), jnp.float32),
        pltpu.VMEM_SHARED((N_PAD,), jnp.float32),
        pltpu.VMEM_SHARED((N_PAD,), jnp.float32),
    ],
)
def _degree_kernel(gi_s3, gi_d3, ds_s3, ds_d3, as_s3, as_d3, zeros1,
                   o_gis, o_gid, o_as, o_dss, o_dsd, o_ad,
                   idx_v, ones_v, acc_a, acc_b, acc_c):
    t = lax.axis_index("s")
    c = lax.axis_index("c")
    sl = pl.ds(t * ROWS_1D, ROWS_1D)
    pltpu.sync_copy(zeros1.at[sl], acc_a.at[sl])
    pltpu.sync_copy(zeros1.at[sl], acc_b.at[sl])
    pltpu.sync_copy(zeros1.at[sl], acc_c.at[sl])
    for i in range(CHUNK // 16):
        ones_v[pl.ds(i * 16, 16)] = jnp.ones((16,), jnp.float32)
    plsc.subcore_barrier()

    def run(idx3, nch, acc):
        pltpu.sync_copy(idx3.at[t], idx_v.at[pl.ds(0, nch)])

        def body(j, carry):
            pltpu.sync_copy(ones_v, acc.at[idx_v.at[j]], add=True)
            return carry

        lax.fori_loop(0, nch, body, 0)

    @pl.when(c == 0)
    def _():
        run(gi_s3, NCH_I, acc_a)
        run(gi_d3, NCH_I, acc_b)
        run(as_s3, NCH_S, acc_c)

    @pl.when(c == 1)
    def _():
        run(ds_s3, NCH_S, acc_a)
        run(ds_d3, NCH_S, acc_b)
        run(as_d3, NCH_S, acc_c)

    plsc.subcore_barrier()

    @pl.when(c == 0)
    def _():
        pltpu.sync_copy(acc_a.at[sl], o_gis.at[sl])
        pltpu.sync_copy(acc_b.at[sl], o_gid.at[sl])
        pltpu.sync_copy(acc_c.at[sl], o_as.at[sl])

    @pl.when(c == 1)
    def _():
        pltpu.sync_copy(acc_a.at[sl], o_dss.at[sl])
        pltpu.sync_copy(acc_b.at[sl], o_dsd.at[sl])
        pltpu.sync_copy(acc_c.at[sl], o_ad.at[sl])


NBLK_DG0 = 1


def _make_scatter_kernel(width):
    @functools.partial(
        pl.kernel,
        out_type=[jax.ShapeDtypeStruct((N_PAD, width), jnp.float32)] * 5,
        mesh=_mesh,
        compiler_params=_sc_params,
        scratch_types=[
            pltpu.VMEM((BLK_CH, CHUNK), jnp.int32),
            pltpu.VMEM((BLK_CH, CHUNK), jnp.int32),
            pltpu.VMEM((CHUNK, width), jnp.float32),
            pltpu.VMEM((CHUNK, width), jnp.float32),
            pltpu.VMEM((CHUNK, width), jnp.float32),
            pltpu.SemaphoreType.DMA,
            pltpu.SemaphoreType.DMA,
            pltpu.VMEM_SHARED((N_PAD, width), jnp.float32),
            pltpu.VMEM_SHARED((N_PAD, width), jnp.float32),
        ],
    )
    def scatter_kernel(hs_ii, hs_dg, hs_dd, hs_gd,
                       gi_s3, gi_d3, ds_s3, ds_d3, as_s3, as_d3,
                       o_ii, o_dga, o_dgb, o_dd, o_gd,
                       src_v, dst_v, rows_a, rows_b, zbuf, sem_a, sem_b,
                       acc0, acc1):
        t = lax.axis_index("s")
        c = lax.axis_index("c")
        rsl = pl.ds(t * ROWS_1D, ROWS_1D)

        def zrow(r, carry):
            for i in range(width // 16):
                zbuf[r, pl.ds(i * 16, 16)] = jnp.zeros((16,), jnp.float32)
            return carry

        lax.fori_loop(0, CHUNK, zrow, 0)

        def zero_acc(acc):
            for k in range(ROWS_1D // CHUNK):
                pltpu.sync_copy(
                    zbuf, acc.at[pl.ds(t * ROWS_1D + k * CHUNK, CHUNK)])

        zero_acc(acc0)
        zero_acc(acc1)
        plsc.subcore_barrier()

        def run(tbl, s3, d3, blk0, nblk, acc):
            def blk_body(b, carry):
                base = (blk0 + b) * BLK_CH
                pltpu.sync_copy(s3.at[t, pl.ds(base, BLK_CH)], src_v)
                pltpu.sync_copy(d3.at[t, pl.ds(base, BLK_CH)], dst_v)
                pltpu.async_copy(tbl.at[src_v.at[0]], rows_a, sem_a)

                def body(k, c2):
                    j = 2 * k
                    jb = j + 1
                    ja_next = jnp.minimum(j + 2, BLK_CH - 2)
                    pltpu.make_async_copy(
                        tbl.at[src_v.at[j]], rows_a, sem_a).wait()
                    pltpu.async_copy(tbl.at[src_v.at[jb]], rows_b, sem_b)
                    pltpu.sync_copy(rows_a, acc.at[dst_v.at[j]], add=True)
                    pltpu.make_async_copy(
                        tbl.at[src_v.at[jb]], rows_b, sem_b).wait()
                    pltpu.async_copy(tbl.at[src_v.at[ja_next]], rows_a, sem_a)
                    pltpu.sync_copy(rows_b, acc.at[dst_v.at[jb]], add=True)
                    return c2

                lax.fori_loop(0, BLK_CH // 2, body, 0)
                pltpu.make_async_copy(tbl.at[src_v.at[0]], rows_a, sem_a).wait()
                return carry

            lax.fori_loop(0, nblk, blk_body, 0)

        @pl.when(c == 0)
        def _():
            run(hs_ii, gi_s3, gi_d3, 0, NCH_I // BLK_CH, acc0)
            run(hs_dg, as_d3, as_s3, 0, NBLK_DG0, acc1)

        @pl.when(c == 1)
        def _():
            run(hs_dd, ds_s3, ds_d3, 0, NCH_S // BLK_CH, acc0)
            plsc.subcore_barrier()
            pltpu.sync_copy(acc0.at[rsl], o_dd.at[rsl])
            zero_acc(acc0)
            plsc.subcore_barrier()
            run(hs_gd, as_s3, as_d3, 0, NCH_S // BLK_CH, acc1)
            run(hs_dg, as_d3, as_s3, NBLK_DG0, NCH_S // BLK_CH - NBLK_DG0, acc0)

        plsc.subcore_barrier()

        @pl.when(c == 0)
        def _():
            pltpu.sync_copy(acc0.at[rsl], o_ii.at[rsl])
            pltpu.sync_copy(acc1.at[rsl], o_dga.at[rsl])

        @pl.when(c == 1)
        def _():
            pltpu.sync_copy(acc1.at[rsl], o_gd.at[rsl])
            pltpu.sync_copy(acc0.at[rsl], o_dgb.at[rsl])

    return scatter_kernel


_scatter64 = _make_scatter_kernel(64)
_scatter32 = _make_scatter_kernel(32)


BLK = 1024
_GRID = N_PAD // BLK


def _rowmm(x, w):
    return lax.dot_general(x, w, (((1,), (1,)), ((), ())),
                           preferred_element_type=jnp.float32)


def _prep1_body(xg, xd, w_ii, w_gd, w_dd, w_dg,
                c_gis, c_as, c_dss, c_ad,
                hs_ii, hs_gd, hs_dd, hs_dg):
    a_gis = lax.rsqrt(c_gis[...] + 1.0)[:, None]
    a_as = lax.rsqrt(jnp.maximum(c_as[...], 1.0))[:, None]
    a_dss = lax.rsqrt(c_dss[...] + 1.0)[:, None]
    a_ad = lax.rsqrt(jnp.maximum(c_ad[...], 1.0))[:, None]
    hs_ii[...] = _rowmm(xg[...], w_ii[...]) * a_gis
    hs_gd[...] = _rowmm(xg[...], w_gd[...]) * a_as
    hs_dd[...] = _rowmm(xd[...], w_dd[...]) * a_dss
    hs_dg[...] = _rowmm(xd[...], w_dg[...]) * a_ad


def _combine_body(m_ii, m_dga, m_dgb, m_dd, m_gd, hs_ii, hs_dd,
                  c_gis, c_gid, c_as, c_dss, c_dsd, c_ad,
                  bias_g, bias_d, w_ii, w_gd, w_dd, w_dg,
                  o_ii, o_gd, o_dd, o_dg):
    a_gis = lax.rsqrt(c_gis[...] + 1.0)[:, None]
    a_gid = lax.rsqrt(c_gid[...] + 1.0)[:, None]
    a_as = lax.rsqrt(jnp.maximum(c_as[...], 1.0))[:, None]
    a_dss = lax.rsqrt(c_dss[...] + 1.0)[:, None]
    a_dsd = lax.rsqrt(c_dsd[...] + 1.0)[:, None]
    a_ad = lax.rsqrt(jnp.maximum(c_ad[...], 1.0))[:, None]
    g = jnp.maximum(a_gid * (m_ii[...] + hs_ii[...])
                    + a_as * (m_dga[...] + m_dgb[...]) + bias_g[...], 0.0)
    d = jnp.maximum(a_dsd * (m_dd[...] + hs_dd[...]) + a_ad * m_gd[...]
                    + bias_d[...], 0.0)
    o_ii[...] = _rowmm(g, w_ii[...]) * a_gis
    o_gd[...] = _rowmm(g, w_gd[...]) * a_as
    o_dd[...] = _rowmm(d, w_dd[...]) * a_dss
    o_dg[...] = _rowmm(d, w_dg[...]) * a_ad


def _final_body(m_ii, m_dga, m_dgb, m_dd, m_gd, hs_ii, hs_dd,
                c_gid, c_as, c_dsd, c_ad, bias_g, bias_d,
                o_g, o_d):
    a_gid = lax.rsqrt(c_gid[...] + 1.0)[:, None]
    a_as = lax.rsqrt(jnp.maximum(c_as[...], 1.0))[:, None]
    a_dsd = lax.rsqrt(c_dsd[...] + 1.0)[:, None]
    a_ad = lax.rsqrt(jnp.maximum(c_ad[...], 1.0))[:, None]
    o_g[...] = jnp.maximum(a_gid * (m_ii[...] + hs_ii[...])
                           + a_as * (m_dga[...] + m_dgb[...]) + bias_g[...], 0.0)
    o_d[...] = jnp.maximum(a_dsd * (m_dd[...] + hs_dd[...]) + a_ad * m_gd[...]
                           + bias_d[...], 0.0)


def _row_spec(width):
    if width is None:
        return pl.BlockSpec((BLK,), lambda i: (i,))
    return pl.BlockSpec((BLK, width), lambda i: (i, 0))


def _full_spec(shape):
    return pl.BlockSpec(shape, lambda i: tuple(0 for _ in shape))


def _tc_call(body, in_arrays, in_specs, out_widths):
    return pl.pallas_call(
        body,
        grid=(_GRID,),
        in_specs=in_specs,
        out_specs=[_row_spec(w) for w in out_widths],
        out_shape=[jax.ShapeDtypeStruct((N_PAD, w), jnp.float32)
                   for w in out_widths],
    )(*in_arrays)


def kernel(x_gene, x_disease, edge_interacts, edge_similar, edge_assoc,
           W1_ii, b1_ii, W1_dd, b1_dd, W1_gd, b1_gd, W1_dg, b1_dg,
           W2_ii, b2_ii, W2_dd, b2_dd, W2_gd, b2_gd, W2_dg, b2_dg):
    pad_rows = N_PAD - N_NODE
    xg = jnp.pad(x_gene.astype(jnp.float32), ((0, pad_rows), (0, 0)))
    xd = jnp.pad(x_disease.astype(jnp.float32), ((0, pad_rows), (0, 0)))

    gi_s3 = _pad3(edge_interacts[0], NCH_I)
    gi_d3 = _pad3(edge_interacts[1], NCH_I)
    ds_s3 = _pad3(edge_similar[0], NCH_S)
    ds_d3 = _pad3(edge_similar[1], NCH_S)
    as_s3 = _pad3(edge_assoc[0], NCH_S)
    as_d3 = _pad3(edge_assoc[1], NCH_S)

    zeros1 = jnp.zeros((N_PAD,), jnp.float32)

    c_gis, c_gid, c_as, c_dss, c_dsd, c_ad = _degree_kernel(
        gi_s3, gi_d3, ds_s3, ds_d3, as_s3, as_d3, zeros1)

    row = _row_spec
    full = _full_spec

    hs1_ii, hs1_gd, hs1_dd, hs1_dg = _tc_call(
        _prep1_body,
        [xg, xd, W1_ii, W1_gd, W1_dd, W1_dg, c_gis, c_as, c_dss, c_ad],
        [row(128), row(128),
         full((64, 128)), full((64, 128)), full((64, 128)), full((64, 128)),
         row(None), row(None), row(None), row(None)],
        [64, 64, 64, 64])

    m1_ii, m1_dga, m1_dgb, m1_dd, m1_gd = _scatter64(
        hs1_ii, hs1_dg, hs1_dd, hs1_gd,
        gi_s3, gi_d3, ds_s3, ds_d3, as_s3, as_d3)

    hs2_ii, hs2_gd, hs2_dd, hs2_dg = _tc_call(
        _combine_body,
        [m1_ii, m1_dga, m1_dgb, m1_dd, m1_gd, hs1_ii, hs1_dd,
         c_gis, c_gid, c_as, c_dss, c_dsd, c_ad,
         (b1_ii + b1_dg).reshape(1, 64), (b1_dd + b1_gd).reshape(1, 64),
         W2_ii, W2_gd, W2_dd, W2_dg],
        [row(64)] * 7 + [row(None)] * 6
        + [full((1, 64)), full((1, 64)),
           full((32, 64)), full((32, 64)), full((32, 64)), full((32, 64))],
        [32, 32, 32, 32])

    m2_ii, m2_dga, m2_dgb, m2_dd, m2_gd = _scatter32(
        hs2_ii, hs2_dg, hs2_dd, hs2_gd,
        gi_s3, gi_d3, ds_s3, ds_d3, as_s3, as_d3)

    g2, d2 = _tc_call(
        _final_body,
        [m2_ii, m2_dga, m2_dgb, m2_dd, m2_gd, hs2_ii, hs2_dd,
         c_gid, c_as, c_dsd, c_ad,
         (b2_ii + b2_dg).reshape(1, 32), (b2_dd + b2_gd).reshape(1, 32)],
        [row(32)] * 7 + [row(None)] * 4 + [full((1, 32)), full((1, 32))],
        [32, 32])

    return (g2[:N_NODE], d2[:N_NODE])


# speedup vs baseline: 33.0364x; 1.3478x over previous
import functools

import jax
import jax.numpy as jnp
from jax import lax
from jax.experimental import pallas as pl
from jax.experimental.pallas import tpu as pltpu
from jax.experimental.pallas import tpu_sc as plsc

N_NODE = 10000
N_PAD = 10240
NC, NS = 2, 16
CHUNK = 128
ROWS_1D = N_PAD // NS
N_SPREAD = 64

E_INT = 320000
E_SIM = 160000
BLK_CH = 40
NCH_I = 160
NCH_S = 80

_mesh = plsc.VectorSubcoreMesh(core_axis_name="c", subcore_axis_name="s")
_sc_params = pltpu.CompilerParams(use_tc_tiling_on_sc=False)


def _pad3(row, nch):
    tot = NS * nch * CHUNK
    pad = tot - row.shape[0]
    fill = N_NODE + (jnp.arange(pad, dtype=jnp.int32) % N_SPREAD)
    return jnp.concatenate([row.astype(jnp.int32), fill]).reshape(NS, nch, CHUNK)


@functools.partial(
    pl.kernel,
    out_type=[jax.ShapeDtypeStruct((N_PAD,), jnp.float32)] * 6,
    mesh=_mesh,
    compiler_params=_sc_params,
    scratch_types=[
        pltpu.VMEM((NCH_I, CHUNK), jnp.int32),
        pltpu.VMEM((CHUNK,), jnp.float32),
        pltpu.SemaphoreType.DMA,
        pltpu.VMEM_SHARED((N_PAD,), jnp.float32),
        pltpu.VMEM_SHARED((N_PAD,), jnp.float32),
        pltpu.VMEM_SHARED((N_PAD,), jnp.float32),
    ],
)
def _degree_kernel(gi_s3, gi_d3, ds_s3, ds_d3, as_s3, as_d3, zeros1,
                   o_gis, o_gid, o_as, o_dss, o_dsd, o_ad,
                   idx_v, ones_v, dsem, acc_a, acc_b, acc_c):
    t = lax.axis_index("s")
    c = lax.axis_index("c")
    sl = pl.ds(t * ROWS_1D, ROWS_1D)
    pltpu.sync_copy(zeros1.at[sl], acc_a.at[sl])
    pltpu.sync_copy(zeros1.at[sl], acc_b.at[sl])
    pltpu.sync_copy(zeros1.at[sl], acc_c.at[sl])
    for i in range(CHUNK // 16):
        ones_v[pl.ds(i * 16, 16)] = jnp.ones((16,), jnp.float32)
    plsc.subcore_barrier()

    def run(idx3, nch, acc):
        pltpu.sync_copy(idx3.at[t], idx_v.at[pl.ds(0, nch)])

        def body(g, carry):
            for i in range(8):
                pltpu.async_copy(
                    ones_v, acc.at[idx_v.at[g * 8 + i]], dsem, add=True)
            for i in range(8):
                pltpu.make_async_copy(
                    ones_v, acc.at[idx_v.at[g * 8 + i]], dsem).wait()
            return carry

        lax.fori_loop(0, nch // 8, body, 0)

    @pl.when(c == 0)
    def _():
        run(gi_s3, NCH_I, acc_a)
        run(gi_d3, NCH_I, acc_b)
        run(as_s3, NCH_S, acc_c)

    @pl.when(c == 1)
    def _():
        run(ds_s3, NCH_S, acc_a)
        run(ds_d3, NCH_S, acc_b)
        run(as_d3, NCH_S, acc_c)

    plsc.subcore_barrier()

    @pl.when(c == 0)
    def _():
        pltpu.sync_copy(acc_a.at[sl], o_gis.at[sl])
        pltpu.sync_copy(acc_b.at[sl], o_gid.at[sl])
        pltpu.sync_copy(acc_c.at[sl], o_as.at[sl])

    @pl.when(c == 1)
    def _():
        pltpu.sync_copy(acc_a.at[sl], o_dss.at[sl])
        pltpu.sync_copy(acc_b.at[sl], o_dsd.at[sl])
        pltpu.sync_copy(acc_c.at[sl], o_ad.at[sl])


NBLK_DG0 = 1


def _make_scatter_kernel(width):
    @functools.partial(
        pl.kernel,
        out_type=[jax.ShapeDtypeStruct((N_PAD, width), jnp.float32)] * 5,
        mesh=_mesh,
        compiler_params=_sc_params,
        scratch_types=[
            pltpu.VMEM((BLK_CH, CHUNK), jnp.int32),
            pltpu.VMEM((BLK_CH, CHUNK), jnp.int32),
            [pltpu.VMEM((CHUNK, width), jnp.float32)] * 4,
            [pltpu.SemaphoreType.DMA] * 4,
            [pltpu.SemaphoreType.DMA] * 4,
            pltpu.VMEM_SHARED((N_PAD, width), jnp.float32),
            pltpu.VMEM_SHARED((N_PAD, width), jnp.float32),
        ],
    )
    def scatter_kernel(hs_ii, hs_dg, hs_dd, hs_gd,
                       gi_s3, gi_d3, ds_s3, ds_d3, as_s3, as_d3, zeros2,
                       o_ii, o_dga, o_dgb, o_dd, o_gd,
                       src_v, dst_v, bufs, gsems, ssems,
                       acc0, acc1):
        t = lax.axis_index("s")
        c = lax.axis_index("c")
        rsl = pl.ds(t * ROWS_1D, ROWS_1D)

        def zero_acc(acc):
            pltpu.sync_copy(zeros2.at[rsl], acc.at[rsl])

        zero_acc(acc0)
        zero_acc(acc1)
        plsc.subcore_barrier()

        def run(tbl, s3, d3, blk0, nblk, acc):
            def blk_body(b, carry):
                base = (blk0 + b) * BLK_CH
                pltpu.sync_copy(s3.at[t, pl.ds(base, BLK_CH)], src_v)
                pltpu.sync_copy(d3.at[t, pl.ds(base, BLK_CH)], dst_v)
                for i in range(4):
                    pltpu.async_copy(tbl.at[src_v.at[i]], bufs[i], gsems[i])

                def body(g, c2):
                    for i in range(4):
                        j = 4 * g + i
                        pltpu.make_async_copy(
                            tbl.at[src_v.at[j]], bufs[i], gsems[i]).wait()
                        pltpu.async_copy(
                            bufs[i], acc.at[dst_v.at[j]], ssems[i], add=True)
                    for i in range(4):
                        j = 4 * g + i
                        jn = jnp.minimum(4 * g + 4 + i, BLK_CH - 4 + i)
                        pltpu.make_async_copy(
                            bufs[i], acc.at[dst_v.at[j]], ssems[i]).wait()
                        pltpu.async_copy(
                            tbl.at[src_v.at[jn]], bufs[i], gsems[i])
                    return c2

                lax.fori_loop(0, BLK_CH // 4, body, 0)
                for i in range(4):
                    pltpu.make_async_copy(
                        tbl.at[src_v.at[BLK_CH - 4 + i]],
                        bufs[i], gsems[i]).wait()
                return carry

            lax.fori_loop(0, nblk, blk_body, 0)

        @pl.when(c == 0)
        def _():
            run(hs_ii, gi_s3, gi_d3, 0, NCH_I // BLK_CH, acc0)
            run(hs_dg, as_d3, as_s3, 0, NBLK_DG0, acc1)

        @pl.when(c == 1)
        def _():
            run(hs_dd, ds_s3, ds_d3, 0, NCH_S // BLK_CH, acc0)
            plsc.subcore_barrier()
            pltpu.sync_copy(acc0.at[rsl], o_dd.at[rsl])
            zero_acc(acc0)
            plsc.subcore_barrier()
            run(hs_gd, as_s3, as_d3, 0, NCH_S // BLK_CH, acc1)
            run(hs_dg, as_d3, as_s3, NBLK_DG0, NCH_S // BLK_CH - NBLK_DG0, acc0)

        plsc.subcore_barrier()

        @pl.when(c == 0)
        def _():
            pltpu.sync_copy(acc0.at[rsl], o_ii.at[rsl])
            pltpu.sync_copy(acc1.at[rsl], o_dga.at[rsl])

        @pl.when(c == 1)
        def _():
            pltpu.sync_copy(acc1.at[rsl], o_gd.at[rsl])
            pltpu.sync_copy(acc0.at[rsl], o_dgb.at[rsl])

    return scatter_kernel


_scatter64 = _make_scatter_kernel(64)
_scatter32 = _make_scatter_kernel(32)


BLK = 1024
_GRID = N_PAD // BLK


def _rowmm(x, w):
    return lax.dot_general(x, w, (((1,), (1,)), ((), ())),
                           preferred_element_type=jnp.float32)


def _prep1_body(xg, xd, w_ii, w_gd, w_dd, w_dg,
                c_gis, c_as, c_dss, c_ad,
                hs_ii, hs_gd, hs_dd, hs_dg):
    a_gis = lax.rsqrt(c_gis[...] + 1.0)[:, None]
    a_as = lax.rsqrt(jnp.maximum(c_as[...], 1.0))[:, None]
    a_dss = lax.rsqrt(c_dss[...] + 1.0)[:, None]
    a_ad = lax.rsqrt(jnp.maximum(c_ad[...], 1.0))[:, None]
    hs_ii[...] = _rowmm(xg[...], w_ii[...]) * a_gis
    hs_gd[...] = _rowmm(xg[...], w_gd[...]) * a_as
    hs_dd[...] = _rowmm(xd[...], w_dd[...]) * a_dss
    hs_dg[...] = _rowmm(xd[...], w_dg[...]) * a_ad


def _combine_body(m_ii, m_dga, m_dgb, m_dd, m_gd, hs_ii, hs_dd,
                  c_gis, c_gid, c_as, c_dss, c_dsd, c_ad,
                  bias_g, bias_d, w_ii, w_gd, w_dd, w_dg,
                  o_ii, o_gd, o_dd, o_dg):
    a_gis = lax.rsqrt(c_gis[...] + 1.0)[:, None]
    a_gid = lax.rsqrt(c_gid[...] + 1.0)[:, None]
    a_as = lax.rsqrt(jnp.maximum(c_as[...], 1.0))[:, None]
    a_dss = lax.rsqrt(c_dss[...] + 1.0)[:, None]
    a_dsd = lax.rsqrt(c_dsd[...] + 1.0)[:, None]
    a_ad = lax.rsqrt(jnp.maximum(c_ad[...], 1.0))[:, None]
    g = jnp.maximum(a_gid * (m_ii[...] + hs_ii[...])
                    + a_as * (m_dga[...] + m_dgb[...]) + bias_g[...], 0.0)
    d = jnp.maximum(a_dsd * (m_dd[...] + hs_dd[...]) + a_ad * m_gd[...]
                    + bias_d[...], 0.0)
    o_ii[...] = _rowmm(g, w_ii[...]) * a_gis
    o_gd[...] = _rowmm(g, w_gd[...]) * a_as
    o_dd[...] = _rowmm(d, w_dd[...]) * a_dss
    o_dg[...] = _rowmm(d, w_dg[...]) * a_ad


def _final_body(m_ii, m_dga, m_dgb, m_dd, m_gd, hs_ii, hs_dd,
                c_gid, c_as, c_dsd, c_ad, bias_g, bias_d,
                o_g, o_d):
    a_gid = lax.rsqrt(c_gid[...] + 1.0)[:, None]
    a_as = lax.rsqrt(jnp.maximum(c_as[...], 1.0))[:, None]
    a_dsd = lax.rsqrt(c_dsd[...] + 1.0)[:, None]
    a_ad = lax.rsqrt(jnp.maximum(c_ad[...], 1.0))[:, None]
    o_g[...] = jnp.maximum(a_gid * (m_ii[...] + hs_ii[...])
                           + a_as * (m_dga[...] + m_dgb[...]) + bias_g[...], 0.0)
    o_d[...] = jnp.maximum(a_dsd * (m_dd[...] + hs_dd[...]) + a_ad * m_gd[...]
                           + bias_d[...], 0.0)


def _row_spec(width):
    if width is None:
        return pl.BlockSpec((BLK,), lambda i: (i,))
    return pl.BlockSpec((BLK, width), lambda i: (i, 0))


def _full_spec(shape):
    return pl.BlockSpec(shape, lambda i: tuple(0 for _ in shape))


def _tc_call(body, in_arrays, in_specs, out_widths):
    return pl.pallas_call(
        body,
        grid=(_GRID,),
        in_specs=in_specs,
        out_specs=[_row_spec(w) for w in out_widths],
        out_shape=[jax.ShapeDtypeStruct((N_PAD, w), jnp.float32)
                   for w in out_widths],
    )(*in_arrays)


def kernel(x_gene, x_disease, edge_interacts, edge_similar, edge_assoc,
           W1_ii, b1_ii, W1_dd, b1_dd, W1_gd, b1_gd, W1_dg, b1_dg,
           W2_ii, b2_ii, W2_dd, b2_dd, W2_gd, b2_gd, W2_dg, b2_dg):
    pad_rows = N_PAD - N_NODE
    xg = jnp.pad(x_gene.astype(jnp.float32), ((0, pad_rows), (0, 0)))
    xd = jnp.pad(x_disease.astype(jnp.float32), ((0, pad_rows), (0, 0)))

    gi_s3 = _pad3(edge_interacts[0], NCH_I)
    gi_d3 = _pad3(edge_interacts[1], NCH_I)
    ds_s3 = _pad3(edge_similar[0], NCH_S)
    ds_d3 = _pad3(edge_similar[1], NCH_S)
    as_s3 = _pad3(edge_assoc[0], NCH_S)
    as_d3 = _pad3(edge_assoc[1], NCH_S)

    zeros1 = jnp.zeros((N_PAD,), jnp.float32)
    zeros64 = jnp.zeros((N_PAD, 64), jnp.float32)
    zeros32 = jnp.zeros((N_PAD, 32), jnp.float32)

    c_gis, c_gid, c_as, c_dss, c_dsd, c_ad = _degree_kernel(
        gi_s3, gi_d3, ds_s3, ds_d3, as_s3, as_d3, zeros1)

    row = _row_spec
    full = _full_spec

    hs1_ii, hs1_gd, hs1_dd, hs1_dg = _tc_call(
        _prep1_body,
        [xg, xd, W1_ii, W1_gd, W1_dd, W1_dg, c_gis, c_as, c_dss, c_ad],
        [row(128), row(128),
         full((64, 128)), full((64, 128)), full((64, 128)), full((64, 128)),
         row(None), row(None), row(None), row(None)],
        [64, 64, 64, 64])

    m1_ii, m1_dga, m1_dgb, m1_dd, m1_gd = _scatter64(
        hs1_ii, hs1_dg, hs1_dd, hs1_gd,
        gi_s3, gi_d3, ds_s3, ds_d3, as_s3, as_d3, zeros64)

    hs2_ii, hs2_gd, hs2_dd, hs2_dg = _tc_call(
        _combine_body,
        [m1_ii, m1_dga, m1_dgb, m1_dd, m1_gd, hs1_ii, hs1_dd,
         c_gis, c_gid, c_as, c_dss, c_dsd, c_ad,
         (b1_ii + b1_dg).reshape(1, 64), (b1_dd + b1_gd).reshape(1, 64),
         W2_ii, W2_gd, W2_dd, W2_dg],
        [row(64)] * 7 + [row(None)] * 6
        + [full((1, 64)), full((1, 64)),
           full((32, 64)), full((32, 64)), full((32, 64)), full((32, 64))],
        [32, 32, 32, 32])

    m2_ii, m2_dga, m2_dgb, m2_dd, m2_gd = _scatter32(
        hs2_ii, hs2_dg, hs2_dd, hs2_gd,
        gi_s3, gi_d3, ds_s3, ds_d3, as_s3, as_d3, zeros32)

    g2, d2 = _tc_call(
        _final_body,
        [m2_ii, m2_dga, m2_dgb, m2_dd, m2_gd, hs2_ii, hs2_dd,
         c_gid, c_as, c_dsd, c_ad,
         (b2_ii + b2_dg).reshape(1, 32), (b2_dd + b2_gd).reshape(1, 32)],
        [row(32)] * 7 + [row(None)] * 4 + [full((1, 32)), full((1, 32))],
        [32, 32])

    return (g2[:N_NODE], d2[:N_NODE])
